# baseline (speedup 1.0000x reference)
import functools
import math

import jax
import jax.numpy as jnp
from jax.experimental import pallas as pl
from jax.experimental.pallas import tpu as pltpu
from jax.experimental.pallas import tpu_sc as plsc

FEADIM = 128
N_HEAD = 4
DH = FEADIM // N_HEAD
FFNDIM = 512
SEQ = 2048
LAMBDAS = 0.85
THRESHOLD = 0.5
NEG = -1e9
EPS = 1e-5

BQ = 512
BK = 256
BW = 256
PTILE = 512
PW = 512

bf16 = jnp.bfloat16


def _ln(x, g, b):
    mu = jnp.mean(x, axis=-1, keepdims=True)
    d = x - mu
    var = jnp.mean(d * d, axis=-1, keepdims=True)
    return d / jnp.sqrt(var + EPS) * g + b


def _bdot(a, b_, dims):
    return jax.lax.dot_general(a.astype(bf16), b_.astype(bf16), (dims, ((), ())),
                               preferred_element_type=jnp.float32)


def _windows_kernel(hs_ref, mask_ref, thr_ref, wise_ref, winid_ref, calw_ref,
                    widg_ref):
    hs = hs_ref[...]
    m = mask_ref[...]
    B, T = hs.shape
    valid = jnp.sum(1.0 - m, axis=1, keepdims=True)
    med = jnp.clip((valid * THRESHOLD + (T - valid)).astype(jnp.int32), 0, T - 1)
    bits = jax.lax.bitcast_convert_type(hs, jnp.int32)
    lo = jnp.zeros((B, 1), jnp.int32)
    for bit in range(30, -1, -1):
        mid = lo + (1 << bit)
        cnt = jnp.sum((bits < mid).astype(jnp.int32), axis=1, keepdims=True)
        lo = jnp.where(cnt <= med, mid, lo)
    thr = jax.lax.bitcast_convert_type(lo, jnp.float32)
    thr_ref[...] = jnp.broadcast_to(thr, thr_ref.shape)
    ge = hs >= thr
    x1 = ge.astype(jnp.float32)
    wise_ref[...] = jnp.where(ge, 1.0, LAMBDAS)
    x2 = jnp.concatenate([x1[:, 1:], 1.0 - x1[:, -1:]], axis=1)
    x3 = x2 - x1
    x3 = jnp.where(x3 == -1.0, 1.0, x3)
    x3s = jnp.concatenate([jnp.zeros((B, 1), jnp.float32), x3[:, :-1]], axis=1)
    x3 = jnp.where(x3s + x3 == 2.0, 0.0, x3)
    lane = jax.lax.broadcasted_iota(jnp.int32, (B, T), 1)
    x3 = jnp.where(lane == T - 1, 1.0, x3)
    nzi = (x3 != 0.0).astype(jnp.float32)
    calw = jnp.sum(nzi, axis=1, keepdims=True)
    calw_ref[...] = jnp.broadcast_to(calw, calw_ref.shape)
    c = nzi
    s = 1
    while s < T:
        c = c + jnp.concatenate(
            [jnp.zeros((B, s), jnp.float32), c[:, :-s]], axis=1)
        s *= 2
    winid = c - nzi
    winid_ref[...] = winid
    rowb = jax.lax.broadcasted_iota(jnp.int32, (B, T), 0)
    widg_ref[...] = winid.astype(jnp.int32) + rowb * T


def _lnqkv_kernel(x_ref, g_ref, b_ref, wq_ref, wk_ref, wv_ref,
                  xn_ref, q_ref, k_ref, v_ref):
    x = x_ref[...]
    xn = _ln(x, g_ref[...], b_ref[...])
    xn_ref[...] = xn
    q_ref[...] = _bdot(xn, wq_ref[...], (((1,), (0,))))
    k_ref[...] = _bdot(xn, wk_ref[...], (((1,), (0,))))
    v_ref[...] = _bdot(xn, wv_ref[...], (((1,), (0,))))


def _attn_range(scr_ref, cs_ref, q, k_ref, v_ref, addmask_fn, kt_lo, kt_hi,
                qweight):
    scale = 1.0 / math.sqrt(DH)
    q_bf = q.astype(bf16)
    outs = []
    for h in range(N_HEAD):
        qh = q_bf[:, h * DH:(h + 1) * DH]

        def body1(kt, mx, h=h, qh=qh):
            koff = pl.multiple_of(kt * BK, BK)
            kh = k_ref[0, pl.ds(koff, BK), h * DH:(h + 1) * DH].astype(bf16)
            s = jax.lax.dot_general(qh, kh, (((1,), (1,)), ((), ())),
                                    preferred_element_type=jnp.float32)
            s = s * scale + addmask_fn(koff)
            scr_ref[:, pl.ds(koff, BK)] = s
            return jnp.maximum(mx, jnp.max(s, axis=1, keepdims=True))

        mx = jax.lax.fori_loop(kt_lo, kt_hi, body1,
                               jnp.full((q.shape[0], 1), -1e30, jnp.float32))

        def body2(kt, l, mx=mx):
            koff = pl.multiple_of(kt * BK, BK)
            p = jnp.exp(scr_ref[:, pl.ds(koff, BK)] - mx)
            scr_ref[:, pl.ds(koff, BK)] = p
            return l + jnp.sum(p, axis=1, keepdims=True)

        l = jax.lax.fori_loop(kt_lo, kt_hi, body2,
                              jnp.zeros((q.shape[0], 1), jnp.float32))
        rl = 1.0 / l

        def body3(kt, ctxh, h=h, rl=rl):
            koff = pl.multiple_of(kt * BK, BK)
            pw = scr_ref[:, pl.ds(koff, BK)] * rl
            vh = v_ref[0, pl.ds(koff, BK), h * DH:(h + 1) * DH].astype(bf16)
            contrib = jnp.sum(pw * qweight, axis=0, keepdims=True)
            cs_ref[0, :, pl.ds(koff, BK)] = (
                cs_ref[0, :, pl.ds(koff, BK)] + contrib)
            return ctxh + jax.lax.dot_general(
                pw.astype(bf16), vh, (((1,), (0,)), ((), ())),
                preferred_element_type=jnp.float32)

        ctxh = jax.lax.fori_loop(
            kt_lo, kt_hi, body3, jnp.zeros((q.shape[0], DH), jnp.float32))
        outs.append(ctxh)
    return jnp.concatenate(outs, axis=1)


def _attn1_kernel(q_ref, k_ref, v_ref, wid_ref, ctx_ref, cs_ref, scr_ref):
    i = pl.program_id(1)

    @pl.when(i == 0)
    def _():
        cs_ref[...] = jnp.zeros_like(cs_ref)

    q = q_ref[0]
    wid = wid_ref[0]
    T = wid.shape[1]
    wq = wid_ref[0, :, pl.ds(i * BQ, BQ)]
    wq_min = jnp.min(wq)
    wq_max = jnp.max(wq)
    tvec = jax.lax.broadcasted_iota(jnp.int32, (1, T), 1)
    lo_t = jnp.min(jnp.where(wid == wq_min, tvec, T))
    hi_t = jnp.max(jnp.where(wid == wq_max, tvec, 0))
    kt_lo = lo_t // BK
    kt_hi = hi_t // BK + 1
    wid_qc = jnp.transpose(wq, (1, 0))

    def addmask(koff):
        wk = wid_ref[0, :, pl.ds(koff, BK)]
        return jnp.where(wid_qc == wk, 0.0, NEG)

    ctx_ref[0] = _attn_range(scr_ref, cs_ref, q, k_ref, v_ref, addmask,
                             kt_lo, kt_hi, 1.0)


def _attn2_kernel(q_ref, k_ref, v_ref, calw_ref, ctx_ref, hh_ref, scr_ref):
    b = pl.program_id(0)
    i = pl.program_id(1)

    @pl.when(i == 0)
    def _():
        hh_ref[...] = jnp.zeros_like(hh_ref)

    calw_all = calw_ref[...]
    rows = jax.lax.broadcasted_iota(jnp.int32, calw_all.shape, 0)
    calw_b = jnp.sum(jnp.where(rows == b, calw_all, 0.0)) / calw_all.shape[1]
    maxw = jnp.max(calw_all)
    T = k_ref.shape[1]

    @pl.when(jnp.float32(i * BQ) >= maxw)
    def _():
        ctx_ref[0] = jnp.zeros_like(ctx_ref[0])

    @pl.when(jnp.float32(i * BQ) < maxw)
    def _():
        q = q_ref[0]
        qidx = (jax.lax.broadcasted_iota(jnp.int32, (BQ, 1), 0)
                .astype(jnp.float32) + i * BQ)
        qw = jnp.where(qidx < maxw, 1.0, 0.0) / (N_HEAD * maxw)
        kt_hi = (calw_b.astype(jnp.int32) + (BK - 1)) // BK

        def addmask(koff):
            ki = (jax.lax.broadcasted_iota(jnp.int32, (1, BK), 1) + koff
                  ).astype(jnp.float32)
            return jnp.where(ki < calw_b, 0.0, NEG)

        ctx_ref[0] = _attn_range(scr_ref, hh_ref, q, k_ref, v_ref, addmask,
                                 0, kt_hi, qw)


def _tail(ctx, res, wo, g1, b1, w1, bf1, w2, bf2, g2, b2):
    a = _bdot(ctx, wo, ((1,), (0,)))
    x = _ln(res + a, g1, b1)
    h = _bdot(jax.nn.relu(_bdot(x, w1, ((1,), (0,))) + bf1), w2,
              ((1,), (0,))) + bf2
    return _ln(x + h, g2, b2)


def _post1_kernel(ctx_ref, res_ref, wise_ref, cs_ref,
                  wo_ref, g1_ref, b1_ref, w1_ref, bf1_ref, w2_ref, bf2_ref,
                  g2_ref, b2_ref, lx_ref, y_ref):
    i = pl.program_id(1)
    xo = _tail(ctx_ref[0], res_ref[0], wo_ref[...], g1_ref[...], b1_ref[...],
               w1_ref[...], bf1_ref[...], w2_ref[...], bf2_ref[...],
               g2_ref[...], b2_ref[...])
    wise = wise_ref[0, :, pl.ds(i * PTILE, PTILE)]
    cs = cs_ref[0, :, pl.ds(i * PTILE, PTILE)]
    wise_c = jnp.transpose(wise, (1, 0))
    score_c = jnp.transpose(cs, (1, 0)) / (N_HEAD * SEQ)
    lx = xo * wise_c
    lx_ref[0] = lx
    y_ref[0] = lx * score_c


def _post2_kernel(ctx_ref, res_ref, calw_ref,
                  wo_ref, g1_ref, b1_ref, w1_ref, bf1_ref, w2_ref, bf2_ref,
                  g2_ref, b2_ref, out_ref):
    i = pl.program_id(1)
    maxw = jnp.max(calw_ref[...])

    @pl.when(jnp.float32(i * PTILE) < maxw)
    def _():
        out_ref[0] = _tail(ctx_ref[0], res_ref[0], wo_ref[...], g1_ref[...],
                           b1_ref[...], w1_ref[...], bf1_ref[...], w2_ref[...],
                           bf2_ref[...], g2_ref[...], b2_ref[...])

    @pl.when(jnp.float32(i * PTILE) >= maxw)
    def _():
        out_ref[0] = jnp.zeros_like(out_ref[0])


def _pool_kernel(y_ref, wid_ref, calw_ref, g_ref, b_ref,
                 wq_ref, wk_ref, wv_ref,
                 pre_ref, q_ref, k_ref, v_ref):
    b = pl.program_id(0)
    i = pl.program_id(1)
    T = y_ref.shape[1]
    calw_all = calw_ref[...]
    rows = jax.lax.broadcasted_iota(jnp.int32, calw_all.shape, 0)
    calw_b = jnp.sum(jnp.where(rows == b, calw_all, 0.0)) / calw_all.shape[1]
    g = g_ref[...]
    bb = b_ref[...]

    @pl.when(jnp.float32(i * PW) < calw_b)
    def _():
        y = y_ref[0]
        wio = (jax.lax.broadcasted_iota(jnp.int32, (PW, T), 0) + i * PW
               ).astype(jnp.float32)
        onehot = jnp.where(wio == wid_ref[0], 1.0, 0.0)
        pooled = _bdot(onehot, y, ((1,), (0,)))
        pre = _ln(pooled, g, bb)
        pre_ref[0] = pre
        q_ref[0] = _bdot(pre, wq_ref[...], ((1,), (0,)))
        k_ref[0] = _bdot(pre, wk_ref[...], ((1,), (0,)))
        v_ref[0] = _bdot(pre, wv_ref[...], ((1,), (0,)))

    @pl.when(jnp.float32(i * PW) >= calw_b)
    def _():
        shp = pre_ref[0].shape
        pre_ref[0] = jnp.broadcast_to(bb, shp)
        q_ref[0] = jnp.broadcast_to(_bdot(bb, wq_ref[...], ((1,), (0,))), shp)
        k_ref[0] = jnp.broadcast_to(_bdot(bb, wk_ref[...], ((1,), (0,))), shp)
        v_ref[0] = jnp.broadcast_to(_bdot(bb, wv_ref[...], ((1,), (0,))), shp)


def _final_kernel(lx_ref, up_ref, wid_ref, hh_ref, cs_ref, calw_ref,
                  data_ref, attn_ref):
    b = pl.program_id(0)
    lx = lx_ref[0]
    wid = wid_ref[0]
    T = lx.shape[0]
    calw_all = calw_ref[...]
    rows = jax.lax.broadcasted_iota(jnp.int32, calw_all.shape, 0)
    calw_b = jnp.sum(jnp.where(rows == b, calw_all, 0.0)) / calw_all.shape[1]
    nwt = (calw_b.astype(jnp.int32) + (BW - 1)) // BW
    wid_c = jnp.transpose(wid, (1, 0))
    data_ref[0] = lx + up_ref[0]

    def body(wt, a2):
        woff = pl.multiple_of(wt * BW, BW)
        wio = (jax.lax.broadcasted_iota(jnp.int32, (1, BW), 1) + woff
               ).astype(jnp.float32)
        oh = jnp.where(wid_c == wio, 1.0, 0.0)
        hht = hh_ref[0, :, pl.ds(woff, BW)]
        return a2 + jnp.sum(oh * hht, axis=1, keepdims=True)

    attn2_up = jax.lax.fori_loop(0, nwt, body,
                                 jnp.zeros((T, 1), jnp.float32))
    cs = cs_ref[0]
    outattn = jnp.transpose(cs, (1, 0)) / (N_HEAD * SEQ) * attn2_up
    mx = jnp.max(outattn, axis=0, keepdims=True)
    p = jnp.exp(outattn - mx)
    attn_c = p / jnp.sum(p, axis=0, keepdims=True)
    attn_ref[0] = jnp.transpose(attn_c, (1, 0))


def _sc_upsample(gx_flat, idx_flat):
    rows_total, d = gx_flat.shape
    info = plsc.get_sparse_core_info()
    nw = info.num_cores * info.num_subcores
    rpw = rows_total // nw
    mesh = plsc.VectorSubcoreMesh(core_axis_name="c", subcore_axis_name="s")

    @functools.partial(
        pl.kernel, mesh=mesh,
        out_type=jax.ShapeDtypeStruct((rows_total, d), jnp.float32),
        scratch_types=[
            pltpu.VMEM((rpw,), jnp.int32),
            pltpu.VMEM((rpw, d), jnp.float32),
            pltpu.SemaphoreType.DMA,
        ],
    )
    def k(table_hbm, idx_hbm, out_hbm, idx_v, rows_v, sem):
        wid = jax.lax.axis_index("s") * info.num_cores + jax.lax.axis_index("c")
        base = wid * rpw
        pltpu.sync_copy(idx_hbm.at[pl.ds(base, rpw)], idx_v)
        pltpu.async_copy(table_hbm.at[idx_v], rows_v, sem).wait()
        pltpu.sync_copy(rows_v, out_hbm.at[pl.ds(base, rpw)])

    return k(gx_flat, idx_flat)


def _r2(a):
    return a.reshape(1, -1)


def kernel(x, mask, haltingscore, params):
    B, T, D = x.shape
    f32 = jnp.float32
    p1, p2 = params['dlwt'], params['dgwt']

    thr_r, wise, winid, calw_r, widg = pl.pallas_call(
        _windows_kernel,
        out_shape=(
            jax.ShapeDtypeStruct((B, 128), f32),
            jax.ShapeDtypeStruct((B, T), f32),
            jax.ShapeDtypeStruct((B, T), f32),
            jax.ShapeDtypeStruct((B, 128), f32),
            jax.ShapeDtypeStruct((B, T), jnp.int32),
        ),
    )(haltingscore.astype(f32), mask.astype(f32))
    thr = thr_r[:, 0]
    wid3 = winid.reshape(B, 1, T)
    wise3 = wise.reshape(B, 1, T)

    xf = x.reshape(B * T, D)
    nflat = (B * T) // PTILE
    flat_spec = pl.BlockSpec((PTILE, D), lambda i: (i, 0))
    w_spec = pl.BlockSpec((D, D), lambda i: (0, 0))
    vec_spec = pl.BlockSpec((1, D), lambda i: (0, 0))
    xn, q1, k1, v1 = pl.pallas_call(
        _lnqkv_kernel,
        grid=(nflat,),
        in_specs=[flat_spec, vec_spec, vec_spec, w_spec, w_spec, w_spec],
        out_specs=(flat_spec,) * 4,
        out_shape=(jax.ShapeDtypeStruct((B * T, D), f32),) * 4,
    )(xf, _r2(params['ln1_g']), _r2(params['ln1_b']),
      p1['Wq'], p1['Wk'], p1['Wv'])

    nq = T // BQ
    qt_spec = pl.BlockSpec((1, BQ, D), lambda b, i: (b, i, 0))
    kv_spec = pl.BlockSpec((1, T, D), lambda b, i: (b, 0, 0))
    row_spec = pl.BlockSpec((1, 1, T), lambda b, i: (b, 0, 0))
    scr = pltpu.VMEM((BQ, T), f32)
    ctx1, cs3 = pl.pallas_call(
        _attn1_kernel,
        grid=(B, nq),
        in_specs=[qt_spec, kv_spec, kv_spec, row_spec],
        out_specs=(qt_spec, row_spec),
        out_shape=(jax.ShapeDtypeStruct((B, T, D), f32),
                   jax.ShapeDtypeStruct((B, 1, T), f32)),
        scratch_shapes=[scr],
    )(q1.reshape(B, T, D), k1.reshape(B, T, D), v1.reshape(B, T, D), wid3)

    nt = T // PTILE
    tt_spec = pl.BlockSpec((1, PTILE, D), lambda b, i: (b, i, 0))
    vec2_spec = pl.BlockSpec((1, D), lambda b, i: (0, 0))
    wff1_spec = pl.BlockSpec((D, FFNDIM), lambda b, i: (0, 0))
    bff1_spec = pl.BlockSpec((1, FFNDIM), lambda b, i: (0, 0))
    wff2_spec = pl.BlockSpec((FFNDIM, D), lambda b, i: (0, 0))
    wo_spec = pl.BlockSpec((D, D), lambda b, i: (0, 0))
    lx, y = pl.pallas_call(
        _post1_kernel,
        grid=(B, nt),
        in_specs=[tt_spec, tt_spec, row_spec, row_spec,
                  wo_spec, vec2_spec, vec2_spec, wff1_spec, bff1_spec,
                  wff2_spec, vec2_spec, vec2_spec, vec2_spec],
        out_specs=(tt_spec, tt_spec),
        out_shape=(jax.ShapeDtypeStruct((B, T, D), f32),) * 2,
    )(ctx1, xn.reshape(B, T, D), wise3, cs3,
      p1['Wo'], _r2(p1['g1']), _r2(p1['b1']), p1['W1'], _r2(p1['bf1']),
      p1['W2'], _r2(p1['bf2']), _r2(p1['g2']), _r2(p1['b2']))

    full_spec = pl.BlockSpec((1, T, D), lambda b: (b, 0, 0))
    row1_spec = pl.BlockSpec((1, 1, T), lambda b: (b, 0, 0))
    nwt = T // PW
    pw_spec = pl.BlockSpec((1, PW, D), lambda b, i: (b, i, 0))
    yfull_spec = pl.BlockSpec((1, T, D), lambda b, i: (b, 0, 0))
    calw2_spec = pl.BlockSpec((B, 128), lambda b, i: (0, 0))
    pre, q2, k2, v2 = pl.pallas_call(
        _pool_kernel,
        grid=(B, nwt),
        in_specs=[yfull_spec, row_spec, calw2_spec, vec2_spec, vec2_spec,
                  wo_spec, wo_spec, wo_spec],
        out_specs=(pw_spec,) * 4,
        out_shape=(jax.ShapeDtypeStruct((B, T, D), f32),) * 4,
    )(y, wid3, calw_r, _r2(params['ln2_g']), _r2(params['ln2_b']),
      p2['Wq'], p2['Wk'], p2['Wv'])

    calw_spec = pl.BlockSpec((B, 128), lambda b, i: (0, 0))
    ctx2, hh3 = pl.pallas_call(
        _attn2_kernel,
        grid=(B, nq),
        in_specs=[qt_spec, kv_spec, kv_spec, calw_spec],
        out_specs=(qt_spec, row_spec),
        out_shape=(jax.ShapeDtypeStruct((B, T, D), f32),
                   jax.ShapeDtypeStruct((B, 1, T), f32)),
        scratch_shapes=[scr],
    )(q2, k2, v2, calw_r)

    gx = pl.pallas_call(
        _post2_kernel,
        grid=(B, nt),
        in_specs=[tt_spec, tt_spec, calw2_spec,
                  wo_spec, vec2_spec, vec2_spec, wff1_spec, bff1_spec,
                  wff2_spec, vec2_spec, vec2_spec, vec2_spec],
        out_specs=tt_spec,
        out_shape=jax.ShapeDtypeStruct((B, T, D), f32),
    )(ctx2, pre, calw_r,
      p2['Wo'], _r2(p2['g1']), _r2(p2['b1']), p2['W1'], _r2(p2['bf1']),
      p2['W2'], _r2(p2['bf2']), _r2(p2['g2']), _r2(p2['b2']))

    up = _sc_upsample(gx.reshape(B * T, D), widg.reshape(B * T))

    calwb_spec = pl.BlockSpec((B, 128), lambda b: (0, 0))
    data, attn3 = pl.pallas_call(
        _final_kernel,
        grid=(B,),
        in_specs=[full_spec, full_spec, row1_spec, row1_spec, row1_spec,
                  calwb_spec],
        out_specs=(full_spec, row1_spec),
        out_shape=(jax.ShapeDtypeStruct((B, T, D), f32),
                   jax.ShapeDtypeStruct((B, 1, T), f32)),
    )(lx, up.reshape(B, T, D), wid3, hh3, cs3, calw_r)

    return data, thr, attn3.reshape(B, T)


# speedup vs baseline: 1.2770x; 1.2770x over previous
import functools
import math

import jax
import jax.numpy as jnp
from jax.experimental import pallas as pl
from jax.experimental.pallas import tpu as pltpu
from jax.experimental.pallas import tpu_sc as plsc

FEADIM = 128
N_HEAD = 4
DH = FEADIM // N_HEAD
FFNDIM = 512
SEQ = 2048
LAMBDAS = 0.85
THRESHOLD = 0.5
NEG = -1e9
EPS = 1e-5

BQ = 512
BK = 512
BW = 256
PTILE = 512
PW = 512

bf16 = jnp.bfloat16


def _ln(x, g, b):
    mu = jnp.mean(x, axis=-1, keepdims=True)
    d = x - mu
    var = jnp.mean(d * d, axis=-1, keepdims=True)
    return d / jnp.sqrt(var + EPS) * g + b


def _bdot(a, b_, dims):
    return jax.lax.dot_general(a.astype(bf16), b_.astype(bf16), (dims, ((), ())),
                               preferred_element_type=jnp.float32)


def _windows_kernel(hs_ref, mask_ref, thr_ref, wise_ref, winid_ref, calw_ref,
                    widg_ref):
    hs = hs_ref[...]
    m = mask_ref[...]
    B, T = hs.shape
    valid = jnp.sum(1.0 - m, axis=1, keepdims=True)
    med = jnp.clip((valid * THRESHOLD + (T - valid)).astype(jnp.int32), 0, T - 1)
    bits = jax.lax.bitcast_convert_type(hs, jnp.int32)
    lo = jnp.zeros((B, 1), jnp.int32)
    for bit in range(30, -1, -1):
        mid = lo + (1 << bit)
        cnt = jnp.sum((bits < mid).astype(jnp.int32), axis=1, keepdims=True)
        lo = jnp.where(cnt <= med, mid, lo)
    thr = jax.lax.bitcast_convert_type(lo, jnp.float32)
    thr_ref[...] = jnp.broadcast_to(thr, thr_ref.shape)
    ge = hs >= thr
    x1 = ge.astype(jnp.float32)
    wise_ref[...] = jnp.where(ge, 1.0, LAMBDAS)
    x2 = jnp.concatenate([x1[:, 1:], 1.0 - x1[:, -1:]], axis=1)
    x3 = x2 - x1
    x3 = jnp.where(x3 == -1.0, 1.0, x3)
    x3s = jnp.concatenate([jnp.zeros((B, 1), jnp.float32), x3[:, :-1]], axis=1)
    x3 = jnp.where(x3s + x3 == 2.0, 0.0, x3)
    lane = jax.lax.broadcasted_iota(jnp.int32, (B, T), 1)
    x3 = jnp.where(lane == T - 1, 1.0, x3)
    nzi = (x3 != 0.0).astype(jnp.float32)
    calw = jnp.sum(nzi, axis=1, keepdims=True)
    calw_ref[...] = jnp.broadcast_to(calw, calw_ref.shape)
    c = nzi
    s = 1
    while s < T:
        c = c + jnp.concatenate(
            [jnp.zeros((B, s), jnp.float32), c[:, :-s]], axis=1)
        s *= 2
    winid = c - nzi
    winid_ref[...] = winid
    rowb = jax.lax.broadcasted_iota(jnp.int32, (B, T), 0)
    widg_ref[...] = winid.astype(jnp.int32) + rowb * T


def _lnqkv_kernel(x_ref, g_ref, b_ref, wq_ref, wk_ref, wv_ref,
                  xn_ref, q_ref, k_ref, v_ref):
    x = x_ref[...]
    xn = _ln(x, g_ref[...], b_ref[...])
    xn_ref[...] = xn
    q_ref[...] = _bdot(xn, wq_ref[...], (((1,), (0,))))
    k_ref[...] = _bdot(xn, wk_ref[...], (((1,), (0,))))
    v_ref[...] = _bdot(xn, wv_ref[...], (((1,), (0,))))


def _attn_range(scr_ref, cs_ref, q, k_ref, v_ref, addmask_fn, kt_lo, kt_hi,
                qweight):
    scale = 1.0 / math.sqrt(DH)
    T = k_ref.shape[1]
    nq = q.shape[0]
    q_bf = q.astype(bf16)
    qs = [q_bf[:, h * DH:(h + 1) * DH] for h in range(N_HEAD)]

    def body1(kt, ms):
        koff = pl.multiple_of(kt * BK, BK)
        am = addmask_fn(koff)
        out = []
        for h in range(N_HEAD):
            kh = k_ref[0, pl.ds(koff, BK), h * DH:(h + 1) * DH].astype(bf16)
            s = jax.lax.dot_general(qs[h], kh, (((1,), (1,)), ((), ())),
                                    preferred_element_type=jnp.float32)
            s = s * scale + am
            scr_ref[:, pl.ds(h * T + koff, BK)] = s
            out.append(jnp.maximum(ms[h], jnp.max(s, axis=1, keepdims=True)))
        return tuple(out)

    ms = jax.lax.fori_loop(
        kt_lo, kt_hi, body1,
        tuple(jnp.full((nq, 1), -1e30, jnp.float32) for _ in range(N_HEAD)))

    def body2(kt, ls):
        koff = pl.multiple_of(kt * BK, BK)
        out = []
        for h in range(N_HEAD):
            p = jnp.exp(scr_ref[:, pl.ds(h * T + koff, BK)] - ms[h])
            scr_ref[:, pl.ds(h * T + koff, BK)] = p
            out.append(ls[h] + jnp.sum(p, axis=1, keepdims=True))
        return tuple(out)

    ls = jax.lax.fori_loop(
        kt_lo, kt_hi, body2,
        tuple(jnp.zeros((nq, 1), jnp.float32) for _ in range(N_HEAD)))
    rls = [1.0 / l for l in ls]

    def body3(kt, ctxs):
        koff = pl.multiple_of(kt * BK, BK)
        out = []
        contrib = jnp.zeros((1, BK), jnp.float32)
        for h in range(N_HEAD):
            pw = scr_ref[:, pl.ds(h * T + koff, BK)] * rls[h]
            vh = v_ref[0, pl.ds(koff, BK), h * DH:(h + 1) * DH].astype(bf16)
            contrib = contrib + jnp.sum(pw * qweight, axis=0, keepdims=True)
            out.append(ctxs[h] + jax.lax.dot_general(
                pw.astype(bf16), vh, (((1,), (0,)), ((), ())),
                preferred_element_type=jnp.float32))
        cs_ref[0, :, pl.ds(koff, BK)] = (
            cs_ref[0, :, pl.ds(koff, BK)] + contrib)
        return tuple(out)

    ctxs = jax.lax.fori_loop(
        kt_lo, kt_hi, body3,
        tuple(jnp.zeros((nq, DH), jnp.float32) for _ in range(N_HEAD)))
    return jnp.concatenate(list(ctxs), axis=1)


def _attn1_kernel(q_ref, k_ref, v_ref, wid_ref, ctx_ref, cs_ref, scr_ref):
    i = pl.program_id(1)

    @pl.when(i == 0)
    def _():
        cs_ref[...] = jnp.zeros_like(cs_ref)

    q = q_ref[0]
    wid = wid_ref[0]
    T = wid.shape[1]
    wq = wid_ref[0, :, pl.ds(i * BQ, BQ)]
    wq_min = jnp.min(wq)
    wq_max = jnp.max(wq)
    tvec = jax.lax.broadcasted_iota(jnp.int32, (1, T), 1)
    lo_t = jnp.min(jnp.where(wid == wq_min, tvec, T))
    hi_t = jnp.max(jnp.where(wid == wq_max, tvec, 0))
    kt_lo = lo_t // BK
    kt_hi = hi_t // BK + 1
    wid_qc = jnp.transpose(wq, (1, 0))

    def addmask(koff):
        wk = wid_ref[0, :, pl.ds(koff, BK)]
        return jnp.where(wid_qc == wk, 0.0, NEG)

    ctx_ref[0] = _attn_range(scr_ref, cs_ref, q, k_ref, v_ref, addmask,
                             kt_lo, kt_hi, 1.0)


def _attn2_kernel(q_ref, k_ref, v_ref, calw_ref, ctx_ref, hh_ref, scr_ref):
    b = pl.program_id(0)
    i = pl.program_id(1)

    @pl.when(i == 0)
    def _():
        hh_ref[...] = jnp.zeros_like(hh_ref)

    calw_all = calw_ref[...]
    rows = jax.lax.broadcasted_iota(jnp.int32, calw_all.shape, 0)
    calw_b = jnp.sum(jnp.where(rows == b, calw_all, 0.0)) / calw_all.shape[1]
    maxw = jnp.max(calw_all)
    T = k_ref.shape[1]

    @pl.when(jnp.float32(i * BQ) >= maxw)
    def _():
        ctx_ref[0] = jnp.zeros_like(ctx_ref[0])

    @pl.when(jnp.float32(i * BQ) < maxw)
    def _():
        q = q_ref[0]
        qidx = (jax.lax.broadcasted_iota(jnp.int32, (BQ, 1), 0)
                .astype(jnp.float32) + i * BQ)
        qw = jnp.where(qidx < maxw, 1.0, 0.0) / (N_HEAD * maxw)
        kt_hi = (calw_b.astype(jnp.int32) + (BK - 1)) // BK

        def addmask(koff):
            ki = (jax.lax.broadcasted_iota(jnp.int32, (1, BK), 1) + koff
                  ).astype(jnp.float32)
            return jnp.where(ki < calw_b, 0.0, NEG)

        ctx_ref[0] = _attn_range(scr_ref, hh_ref, q, k_ref, v_ref, addmask,
                                 0, kt_hi, qw)


def _tail(ctx, res, wo, g1, b1, w1, bf1, w2, bf2, g2, b2):
    a = _bdot(ctx, wo, ((1,), (0,)))
    x = _ln(res + a, g1, b1)
    h = _bdot(jax.nn.relu(_bdot(x, w1, ((1,), (0,))) + bf1), w2,
              ((1,), (0,))) + bf2
    return _ln(x + h, g2, b2)


def _post1_kernel(ctx_ref, res_ref, wise_ref, cs_ref,
                  wo_ref, g1_ref, b1_ref, w1_ref, bf1_ref, w2_ref, bf2_ref,
                  g2_ref, b2_ref, lx_ref, y_ref):
    i = pl.program_id(1)
    xo = _tail(ctx_ref[0], res_ref[0], wo_ref[...], g1_ref[...], b1_ref[...],
               w1_ref[...], bf1_ref[...], w2_ref[...], bf2_ref[...],
               g2_ref[...], b2_ref[...])
    wise = wise_ref[0, :, pl.ds(i * PTILE, PTILE)]
    cs = cs_ref[0, :, pl.ds(i * PTILE, PTILE)]
    wise_c = jnp.transpose(wise, (1, 0))
    score_c = jnp.transpose(cs, (1, 0)) / (N_HEAD * SEQ)
    lx = xo * wise_c
    lx_ref[0] = lx
    y_ref[0] = lx * score_c


def _post2_kernel(ctx_ref, res_ref, calw_ref,
                  wo_ref, g1_ref, b1_ref, w1_ref, bf1_ref, w2_ref, bf2_ref,
                  g2_ref, b2_ref, out_ref):
    i = pl.program_id(1)
    maxw = jnp.max(calw_ref[...])

    @pl.when(jnp.float32(i * PTILE) < maxw)
    def _():
        out_ref[0] = _tail(ctx_ref[0], res_ref[0], wo_ref[...], g1_ref[...],
                           b1_ref[...], w1_ref[...], bf1_ref[...], w2_ref[...],
                           bf2_ref[...], g2_ref[...], b2_ref[...])

    @pl.when(jnp.float32(i * PTILE) >= maxw)
    def _():
        out_ref[0] = jnp.zeros_like(out_ref[0])


def _pool_kernel(y_ref, wid_ref, calw_ref, g_ref, b_ref,
                 wq_ref, wk_ref, wv_ref,
                 pre_ref, q_ref, k_ref, v_ref):
    b = pl.program_id(0)
    i = pl.program_id(1)
    T = y_ref.shape[1]
    calw_all = calw_ref[...]
    rows = jax.lax.broadcasted_iota(jnp.int32, calw_all.shape, 0)
    calw_b = jnp.sum(jnp.where(rows == b, calw_all, 0.0)) / calw_all.shape[1]
    g = g_ref[...]
    bb = b_ref[...]

    @pl.when(jnp.float32(i * PW) < calw_b)
    def _():
        y = y_ref[0]
        wio = (jax.lax.broadcasted_iota(jnp.int32, (PW, T), 0) + i * PW
               ).astype(jnp.float32)
        onehot = jnp.where(wio == wid_ref[0], 1.0, 0.0)
        pooled = _bdot(onehot, y, ((1,), (0,)))
        pre = _ln(pooled, g, bb)
        pre_ref[0] = pre
        q_ref[0] = _bdot(pre, wq_ref[...], ((1,), (0,)))
        k_ref[0] = _bdot(pre, wk_ref[...], ((1,), (0,)))
        v_ref[0] = _bdot(pre, wv_ref[...], ((1,), (0,)))

    @pl.when(jnp.float32(i * PW) >= calw_b)
    def _():
        shp = pre_ref[0].shape
        pre_ref[0] = jnp.broadcast_to(bb, shp)
        q_ref[0] = jnp.broadcast_to(_bdot(bb, wq_ref[...], ((1,), (0,))), shp)
        k_ref[0] = jnp.broadcast_to(_bdot(bb, wk_ref[...], ((1,), (0,))), shp)
        v_ref[0] = jnp.broadcast_to(_bdot(bb, wv_ref[...], ((1,), (0,))), shp)


def _final_kernel(lx_ref, up_ref, wid_ref, hh_ref, cs_ref, calw_ref,
                  data_ref, attn_ref):
    b = pl.program_id(0)
    lx = lx_ref[0]
    wid = wid_ref[0]
    T = lx.shape[0]
    calw_all = calw_ref[...]
    rows = jax.lax.broadcasted_iota(jnp.int32, calw_all.shape, 0)
    calw_b = jnp.sum(jnp.where(rows == b, calw_all, 0.0)) / calw_all.shape[1]
    nwt = (calw_b.astype(jnp.int32) + (BW - 1)) // BW
    wid_c = jnp.transpose(wid, (1, 0))
    data_ref[0] = lx + up_ref[0]

    def body(wt, a2):
        woff = pl.multiple_of(wt * BW, BW)
        wio = (jax.lax.broadcasted_iota(jnp.int32, (1, BW), 1) + woff
               ).astype(jnp.float32)
        oh = jnp.where(wid_c == wio, 1.0, 0.0)
        hht = hh_ref[0, :, pl.ds(woff, BW)]
        return a2 + jnp.sum(oh * hht, axis=1, keepdims=True)

    attn2_up = jax.lax.fori_loop(0, nwt, body,
                                 jnp.zeros((T, 1), jnp.float32))
    cs = cs_ref[0]
    outattn = jnp.transpose(cs, (1, 0)) / (N_HEAD * SEQ) * attn2_up
    mx = jnp.max(outattn, axis=0, keepdims=True)
    p = jnp.exp(outattn - mx)
    attn_c = p / jnp.sum(p, axis=0, keepdims=True)
    attn_ref[0] = jnp.transpose(attn_c, (1, 0))


def _sc_upsample(gx_flat, idx_flat):
    rows_total, d = gx_flat.shape
    info = plsc.get_sparse_core_info()
    nw = info.num_cores * info.num_subcores
    rpw = rows_total // nw
    mesh = plsc.VectorSubcoreMesh(core_axis_name="c", subcore_axis_name="s")

    @functools.partial(
        pl.kernel, mesh=mesh,
        out_type=jax.ShapeDtypeStruct((rows_total, d), jnp.float32),
        scratch_types=[
            pltpu.VMEM((rpw,), jnp.int32),
            pltpu.VMEM((rpw, d), jnp.float32),
            pltpu.SemaphoreType.DMA,
        ],
    )
    def k(table_hbm, idx_hbm, out_hbm, idx_v, rows_v, sem):
        wid = jax.lax.axis_index("s") * info.num_cores + jax.lax.axis_index("c")
        base = wid * rpw
        pltpu.sync_copy(idx_hbm.at[pl.ds(base, rpw)], idx_v)
        pltpu.async_copy(table_hbm.at[idx_v], rows_v, sem).wait()
        pltpu.sync_copy(rows_v, out_hbm.at[pl.ds(base, rpw)])

    return k(gx_flat, idx_flat)


def _r2(a):
    return a.reshape(1, -1)


def kernel(x, mask, haltingscore, params):
    B, T, D = x.shape
    f32 = jnp.float32
    p1, p2 = params['dlwt'], params['dgwt']

    thr_r, wise, winid, calw_r, widg = pl.pallas_call(
        _windows_kernel,
        out_shape=(
            jax.ShapeDtypeStruct((B, 128), f32),
            jax.ShapeDtypeStruct((B, T), f32),
            jax.ShapeDtypeStruct((B, T), f32),
            jax.ShapeDtypeStruct((B, 128), f32),
            jax.ShapeDtypeStruct((B, T), jnp.int32),
        ),
    )(haltingscore.astype(f32), mask.astype(f32))
    thr = thr_r[:, 0]
    wid3 = winid.reshape(B, 1, T)
    wise3 = wise.reshape(B, 1, T)

    xf = x.reshape(B * T, D)
    nflat = (B * T) // PTILE
    flat_spec = pl.BlockSpec((PTILE, D), lambda i: (i, 0))
    w_spec = pl.BlockSpec((D, D), lambda i: (0, 0))
    vec_spec = pl.BlockSpec((1, D), lambda i: (0, 0))
    xn, q1, k1, v1 = pl.pallas_call(
        _lnqkv_kernel,
        grid=(nflat,),
        in_specs=[flat_spec, vec_spec, vec_spec, w_spec, w_spec, w_spec],
        out_specs=(flat_spec,) * 4,
        out_shape=(jax.ShapeDtypeStruct((B * T, D), f32),) * 4,
    )(xf, _r2(params['ln1_g']), _r2(params['ln1_b']),
      p1['Wq'], p1['Wk'], p1['Wv'])

    nq = T // BQ
    qt_spec = pl.BlockSpec((1, BQ, D), lambda b, i: (b, i, 0))
    kv_spec = pl.BlockSpec((1, T, D), lambda b, i: (b, 0, 0))
    row_spec = pl.BlockSpec((1, 1, T), lambda b, i: (b, 0, 0))
    scr = pltpu.VMEM((BQ, N_HEAD * T), f32)
    ctx1, cs3 = pl.pallas_call(
        _attn1_kernel,
        grid=(B, nq),
        in_specs=[qt_spec, kv_spec, kv_spec, row_spec],
        out_specs=(qt_spec, row_spec),
        out_shape=(jax.ShapeDtypeStruct((B, T, D), f32),
                   jax.ShapeDtypeStruct((B, 1, T), f32)),
        scratch_shapes=[scr],
    )(q1.reshape(B, T, D), k1.reshape(B, T, D), v1.reshape(B, T, D), wid3)

    nt = T // PTILE
    tt_spec = pl.BlockSpec((1, PTILE, D), lambda b, i: (b, i, 0))
    vec2_spec = pl.BlockSpec((1, D), lambda b, i: (0, 0))
    wff1_spec = pl.BlockSpec((D, FFNDIM), lambda b, i: (0, 0))
    bff1_spec = pl.BlockSpec((1, FFNDIM), lambda b, i: (0, 0))
    wff2_spec = pl.BlockSpec((FFNDIM, D), lambda b, i: (0, 0))
    wo_spec = pl.BlockSpec((D, D), lambda b, i: (0, 0))
    lx, y = pl.pallas_call(
        _post1_kernel,
        grid=(B, nt),
        in_specs=[tt_spec, tt_spec, row_spec, row_spec,
                  wo_spec, vec2_spec, vec2_spec, wff1_spec, bff1_spec,
                  wff2_spec, vec2_spec, vec2_spec, vec2_spec],
        out_specs=(tt_spec, tt_spec),
        out_shape=(jax.ShapeDtypeStruct((B, T, D), f32),) * 2,
    )(ctx1, xn.reshape(B, T, D), wise3, cs3,
      p1['Wo'], _r2(p1['g1']), _r2(p1['b1']), p1['W1'], _r2(p1['bf1']),
      p1['W2'], _r2(p1['bf2']), _r2(p1['g2']), _r2(p1['b2']))

    full_spec = pl.BlockSpec((1, T, D), lambda b: (b, 0, 0))
    row1_spec = pl.BlockSpec((1, 1, T), lambda b: (b, 0, 0))
    nwt = T // PW
    pw_spec = pl.BlockSpec((1, PW, D), lambda b, i: (b, i, 0))
    yfull_spec = pl.BlockSpec((1, T, D), lambda b, i: (b, 0, 0))
    calw2_spec = pl.BlockSpec((B, 128), lambda b, i: (0, 0))
    pre, q2, k2, v2 = pl.pallas_call(
        _pool_kernel,
        grid=(B, nwt),
        in_specs=[yfull_spec, row_spec, calw2_spec, vec2_spec, vec2_spec,
                  wo_spec, wo_spec, wo_spec],
        out_specs=(pw_spec,) * 4,
        out_shape=(jax.ShapeDtypeStruct((B, T, D), f32),) * 4,
    )(y, wid3, calw_r, _r2(params['ln2_g']), _r2(params['ln2_b']),
      p2['Wq'], p2['Wk'], p2['Wv'])

    calw_spec = pl.BlockSpec((B, 128), lambda b, i: (0, 0))
    ctx2, hh3 = pl.pallas_call(
        _attn2_kernel,
        grid=(B, nq),
        in_specs=[qt_spec, kv_spec, kv_spec, calw_spec],
        out_specs=(qt_spec, row_spec),
        out_shape=(jax.ShapeDtypeStruct((B, T, D), f32),
                   jax.ShapeDtypeStruct((B, 1, T), f32)),
        scratch_shapes=[scr],
    )(q2, k2, v2, calw_r)

    gx = pl.pallas_call(
        _post2_kernel,
        grid=(B, nt),
        in_specs=[tt_spec, tt_spec, calw2_spec,
                  wo_spec, vec2_spec, vec2_spec, wff1_spec, bff1_spec,
                  wff2_spec, vec2_spec, vec2_spec, vec2_spec],
        out_specs=tt_spec,
        out_shape=jax.ShapeDtypeStruct((B, T, D), f32),
    )(ctx2, pre, calw_r,
      p2['Wo'], _r2(p2['g1']), _r2(p2['b1']), p2['W1'], _r2(p2['bf1']),
      p2['W2'], _r2(p2['bf2']), _r2(p2['g2']), _r2(p2['b2']))

    up = _sc_upsample(gx.reshape(B * T, D), widg.reshape(B * T))

    calwb_spec = pl.BlockSpec((B, 128), lambda b: (0, 0))
    data, attn3 = pl.pallas_call(
        _final_kernel,
        grid=(B,),
        in_specs=[full_spec, full_spec, row1_spec, row1_spec, row1_spec,
                  calwb_spec],
        out_specs=(full_spec, row1_spec),
        out_shape=(jax.ShapeDtypeStruct((B, T, D), f32),
                   jax.ShapeDtypeStruct((B, 1, T), f32)),
    )(lx, up.reshape(B, T, D), wid3, hh3, cs3, calw_r)

    return data, thr, attn3.reshape(B, T)


# speedup vs baseline: 1.5592x; 1.2210x over previous
import functools
import math

import jax
import jax.numpy as jnp
from jax.experimental import pallas as pl
from jax.experimental.pallas import tpu as pltpu

FEADIM = 128
N_HEAD = 4
DH = FEADIM // N_HEAD
FFNDIM = 512
SEQ = 2048
LAMBDAS = 0.85
THRESHOLD = 0.5
NEG = -1e9
EPS = 1e-5

BQ = 512
BK = 512
BW = 256
PTILE = 512
PW = 512

bf16 = jnp.bfloat16


def _ln(x, g, b):
    mu = jnp.mean(x, axis=-1, keepdims=True)
    d = x - mu
    var = jnp.mean(d * d, axis=-1, keepdims=True)
    return d / jnp.sqrt(var + EPS) * g + b


def _bdot(a, b_, dims):
    return jax.lax.dot_general(a.astype(bf16), b_.astype(bf16), (dims, ((), ())),
                               preferred_element_type=jnp.float32)


def _windows_kernel(hs_ref, mask_ref, thr_ref, wise_ref, winid_ref, calw_ref):
    hs = hs_ref[...]
    m = mask_ref[...]
    B, T = hs.shape
    valid = jnp.sum(1.0 - m, axis=1, keepdims=True)
    med = jnp.clip((valid * THRESHOLD + (T - valid)).astype(jnp.int32), 0, T - 1)
    bits = jax.lax.bitcast_convert_type(hs, jnp.int32)
    lo = jnp.zeros((B, 1), jnp.int32)
    for bit in range(30, -1, -1):
        mid = lo + (1 << bit)
        cnt = jnp.sum((bits < mid).astype(jnp.int32), axis=1, keepdims=True)
        lo = jnp.where(cnt <= med, mid, lo)
    thr = jax.lax.bitcast_convert_type(lo, jnp.float32)
    thr_ref[...] = jnp.broadcast_to(thr, thr_ref.shape)
    ge = hs >= thr
    x1 = ge.astype(jnp.float32)
    wise_ref[...] = jnp.where(ge, 1.0, LAMBDAS)
    x2 = jnp.concatenate([x1[:, 1:], 1.0 - x1[:, -1:]], axis=1)
    x3 = x2 - x1
    x3 = jnp.where(x3 == -1.0, 1.0, x3)
    x3s = jnp.concatenate([jnp.zeros((B, 1), jnp.float32), x3[:, :-1]], axis=1)
    x3 = jnp.where(x3s + x3 == 2.0, 0.0, x3)
    lane = jax.lax.broadcasted_iota(jnp.int32, (B, T), 1)
    x3 = jnp.where(lane == T - 1, 1.0, x3)
    nzi = (x3 != 0.0).astype(jnp.float32)
    calw = jnp.sum(nzi, axis=1, keepdims=True)
    calw_ref[...] = jnp.broadcast_to(calw, calw_ref.shape)
    c = nzi
    s = 1
    while s < T:
        c = c + jnp.concatenate(
            [jnp.zeros((B, s), jnp.float32), c[:, :-s]], axis=1)
        s *= 2
    winid_ref[...] = c - nzi


def _lnqkv_kernel(x_ref, g_ref, b_ref, wq_ref, wk_ref, wv_ref,
                  xn_ref, q_ref, k_ref, v_ref):
    x = x_ref[...]
    xn = _ln(x, g_ref[...], b_ref[...])
    xn_ref[...] = xn
    q_ref[...] = _bdot(xn, wq_ref[...], (((1,), (0,))))
    k_ref[...] = _bdot(xn, wk_ref[...], (((1,), (0,))))
    v_ref[...] = _bdot(xn, wv_ref[...], (((1,), (0,))))


def _attn_range(scr_ref, cs_ref, q, k_ref, v_ref, addmask_fn, kt_lo, kt_hi,
                qweight):
    scale = 1.0 / math.sqrt(DH)
    T = k_ref.shape[1]
    nq = q.shape[0]
    q_bf = q.astype(bf16)
    qs = [q_bf[:, h * DH:(h + 1) * DH] for h in range(N_HEAD)]

    def body1(kt, ms):
        koff = pl.multiple_of(kt * BK, BK)
        am = addmask_fn(koff)
        out = []
        for h in range(N_HEAD):
            kh = k_ref[0, pl.ds(koff, BK), h * DH:(h + 1) * DH].astype(bf16)
            s = jax.lax.dot_general(qs[h], kh, (((1,), (1,)), ((), ())),
                                    preferred_element_type=jnp.float32)
            s = s * scale + am
            scr_ref[:, pl.ds(h * T + koff, BK)] = s
            out.append(jnp.maximum(ms[h], jnp.max(s, axis=1, keepdims=True)))
        return tuple(out)

    ms = jax.lax.fori_loop(
        kt_lo, kt_hi, body1,
        tuple(jnp.full((nq, 1), -1e30, jnp.float32) for _ in range(N_HEAD)))

    def body2(kt, ls):
        koff = pl.multiple_of(kt * BK, BK)
        out = []
        for h in range(N_HEAD):
            p = jnp.exp(scr_ref[:, pl.ds(h * T + koff, BK)] - ms[h])
            scr_ref[:, pl.ds(h * T + koff, BK)] = p
            out.append(ls[h] + jnp.sum(p, axis=1, keepdims=True))
        return tuple(out)

    ls = jax.lax.fori_loop(
        kt_lo, kt_hi, body2,
        tuple(jnp.zeros((nq, 1), jnp.float32) for _ in range(N_HEAD)))
    rls = [1.0 / l for l in ls]

    def body3(kt, ctxs):
        koff = pl.multiple_of(kt * BK, BK)
        out = []
        contrib = jnp.zeros((1, BK), jnp.float32)
        for h in range(N_HEAD):
            pw = scr_ref[:, pl.ds(h * T + koff, BK)] * rls[h]
            vh = v_ref[0, pl.ds(koff, BK), h * DH:(h + 1) * DH].astype(bf16)
            contrib = contrib + jnp.sum(pw * qweight, axis=0, keepdims=True)
            out.append(ctxs[h] + jax.lax.dot_general(
                pw.astype(bf16), vh, (((1,), (0,)), ((), ())),
                preferred_element_type=jnp.float32))
        cs_ref[0, :, pl.ds(koff, BK)] = (
            cs_ref[0, :, pl.ds(koff, BK)] + contrib)
        return tuple(out)

    ctxs = jax.lax.fori_loop(
        kt_lo, kt_hi, body3,
        tuple(jnp.zeros((nq, DH), jnp.float32) for _ in range(N_HEAD)))
    return jnp.concatenate(list(ctxs), axis=1)


def _attn1_kernel(q_ref, k_ref, v_ref, wid_ref, ctx_ref, cs_ref, scr_ref):
    i = pl.program_id(1)

    @pl.when(i == 0)
    def _():
        cs_ref[...] = jnp.zeros_like(cs_ref)

    q = q_ref[0]
    wid = wid_ref[0]
    T = wid.shape[1]
    wq = wid_ref[0, :, pl.ds(i * BQ, BQ)]
    wq_min = jnp.min(wq)
    wq_max = jnp.max(wq)
    tvec = jax.lax.broadcasted_iota(jnp.int32, (1, T), 1)
    lo_t = jnp.min(jnp.where(wid == wq_min, tvec, T))
    hi_t = jnp.max(jnp.where(wid == wq_max, tvec, 0))
    kt_lo = lo_t // BK
    kt_hi = hi_t // BK + 1
    wid_qc = jnp.transpose(wq, (1, 0))

    def addmask(koff):
        wk = wid_ref[0, :, pl.ds(koff, BK)]
        return jnp.where(wid_qc == wk, 0.0, NEG)

    ctx_ref[0] = _attn_range(scr_ref, cs_ref, q, k_ref, v_ref, addmask,
                             kt_lo, kt_hi, 1.0)


def _attn2_kernel(q_ref, k_ref, v_ref, calw_ref, ctx_ref, hh_ref, scr_ref):
    b = pl.program_id(0)
    i = pl.program_id(1)

    @pl.when(i == 0)
    def _():
        hh_ref[...] = jnp.zeros_like(hh_ref)

    calw_all = calw_ref[...]
    rows = jax.lax.broadcasted_iota(jnp.int32, calw_all.shape, 0)
    calw_b = jnp.sum(jnp.where(rows == b, calw_all, 0.0)) / calw_all.shape[1]
    maxw = jnp.max(calw_all)
    T = k_ref.shape[1]

    @pl.when(jnp.float32(i * BQ) >= maxw)
    def _():
        ctx_ref[0] = jnp.zeros_like(ctx_ref[0])

    @pl.when(jnp.float32(i * BQ) < maxw)
    def _():
        q = q_ref[0]
        qidx = (jax.lax.broadcasted_iota(jnp.int32, (BQ, 1), 0)
                .astype(jnp.float32) + i * BQ)
        qw = jnp.where(qidx < maxw, 1.0, 0.0) / (N_HEAD * maxw)
        kt_hi = (calw_b.astype(jnp.int32) + (BK - 1)) // BK

        def addmask(koff):
            ki = (jax.lax.broadcasted_iota(jnp.int32, (1, BK), 1) + koff
                  ).astype(jnp.float32)
            return jnp.where(ki < calw_b, 0.0, NEG)

        ctx_ref[0] = _attn_range(scr_ref, hh_ref, q, k_ref, v_ref, addmask,
                                 0, kt_hi, qw)


def _tail(ctx, res, wo, g1, b1, w1, bf1, w2, bf2, g2, b2):
    a = _bdot(ctx, wo, ((1,), (0,)))
    x = _ln(res + a, g1, b1)
    h = _bdot(jax.nn.relu(_bdot(x, w1, ((1,), (0,))) + bf1), w2,
              ((1,), (0,))) + bf2
    return _ln(x + h, g2, b2)


def _post1_kernel(ctx_ref, res_ref, wise_ref, cs_ref,
                  wo_ref, g1_ref, b1_ref, w1_ref, bf1_ref, w2_ref, bf2_ref,
                  g2_ref, b2_ref, lx_ref, y_ref):
    i = pl.program_id(1)
    xo = _tail(ctx_ref[0], res_ref[0], wo_ref[...], g1_ref[...], b1_ref[...],
               w1_ref[...], bf1_ref[...], w2_ref[...], bf2_ref[...],
               g2_ref[...], b2_ref[...])
    wise = wise_ref[0, :, pl.ds(i * PTILE, PTILE)]
    cs = cs_ref[0, :, pl.ds(i * PTILE, PTILE)]
    wise_c = jnp.transpose(wise, (1, 0))
    score_c = jnp.transpose(cs, (1, 0)) / (N_HEAD * SEQ)
    lx = xo * wise_c
    lx_ref[0] = lx
    y_ref[0] = lx * score_c


def _post2_kernel(ctx_ref, res_ref, calw_ref,
                  wo_ref, g1_ref, b1_ref, w1_ref, bf1_ref, w2_ref, bf2_ref,
                  g2_ref, b2_ref, out_ref):
    i = pl.program_id(1)
    maxw = jnp.max(calw_ref[...])

    @pl.when(jnp.float32(i * PTILE) < maxw)
    def _():
        out_ref[0] = _tail(ctx_ref[0], res_ref[0], wo_ref[...], g1_ref[...],
                           b1_ref[...], w1_ref[...], bf1_ref[...], w2_ref[...],
                           bf2_ref[...], g2_ref[...], b2_ref[...])

    @pl.when(jnp.float32(i * PTILE) >= maxw)
    def _():
        out_ref[0] = jnp.zeros_like(out_ref[0])


def _pool_kernel(y_ref, wid_ref, calw_ref, g_ref, b_ref,
                 wq_ref, wk_ref, wv_ref,
                 pre_ref, q_ref, k_ref, v_ref):
    b = pl.program_id(0)
    i = pl.program_id(1)
    T = y_ref.shape[1]
    calw_all = calw_ref[...]
    rows = jax.lax.broadcasted_iota(jnp.int32, calw_all.shape, 0)
    calw_b = jnp.sum(jnp.where(rows == b, calw_all, 0.0)) / calw_all.shape[1]
    g = g_ref[...]
    bb = b_ref[...]

    @pl.when(jnp.float32(i * PW) < calw_b)
    def _():
        y = y_ref[0]
        wio = (jax.lax.broadcasted_iota(jnp.int32, (PW, T), 0) + i * PW
               ).astype(jnp.float32)
        onehot = jnp.where(wio == wid_ref[0], 1.0, 0.0)
        pooled = _bdot(onehot, y, ((1,), (0,)))
        pre = _ln(pooled, g, bb)
        pre_ref[0] = pre
        q_ref[0] = _bdot(pre, wq_ref[...], ((1,), (0,)))
        k_ref[0] = _bdot(pre, wk_ref[...], ((1,), (0,)))
        v_ref[0] = _bdot(pre, wv_ref[...], ((1,), (0,)))

    @pl.when(jnp.float32(i * PW) >= calw_b)
    def _():
        shp = pre_ref[0].shape
        pre_ref[0] = jnp.broadcast_to(bb, shp)
        q_ref[0] = jnp.broadcast_to(_bdot(bb, wq_ref[...], ((1,), (0,))), shp)
        k_ref[0] = jnp.broadcast_to(_bdot(bb, wk_ref[...], ((1,), (0,))), shp)
        v_ref[0] = jnp.broadcast_to(_bdot(bb, wv_ref[...], ((1,), (0,))), shp)


def _final_kernel(lx_ref, gx_ref, wid_ref, hh_ref, cs_ref, calw_ref,
                  data_ref, attn_ref, acc_ref):
    b = pl.program_id(0)
    lx = lx_ref[0]
    wid = wid_ref[0]
    T = lx.shape[0]
    calw_all = calw_ref[...]
    rows = jax.lax.broadcasted_iota(jnp.int32, calw_all.shape, 0)
    calw_b = jnp.sum(jnp.where(rows == b, calw_all, 0.0)) / calw_all.shape[1]
    nwt = (calw_b.astype(jnp.int32) + (BW - 1)) // BW
    wid_c = jnp.transpose(wid, (1, 0))
    acc_ref[...] = jnp.zeros_like(acc_ref)

    def body(wt, _):
        woff = pl.multiple_of(wt * BW, BW)
        wio = (jax.lax.broadcasted_iota(jnp.int32, (1, BW), 1) + woff
               ).astype(jnp.float32)
        oh = jnp.where(wid_c == wio, 1.0, 0.0)
        gxt = gx_ref[0, pl.ds(woff, BW), :]
        hht = hh_ref[0, :, pl.ds(woff, BW)]
        aug = jnp.concatenate(
            [gxt, jnp.transpose(hht, (1, 0)),
             jnp.zeros((BW, FEADIM - 1), jnp.float32)], axis=1)
        acc_ref[...] = acc_ref[...] + _bdot(oh, aug, ((1,), (0,)))
        return 0

    jax.lax.fori_loop(0, nwt, body, 0)
    acc = acc_ref[...]
    data_ref[0] = lx + acc[:, :FEADIM]
    attn2_up = acc[:, FEADIM:FEADIM + 1]
    cs = cs_ref[0]
    outattn = jnp.transpose(cs, (1, 0)) / (N_HEAD * SEQ) * attn2_up
    mx = jnp.max(outattn, axis=0, keepdims=True)
    p = jnp.exp(outattn - mx)
    attn_c = p / jnp.sum(p, axis=0, keepdims=True)
    attn_ref[0] = jnp.transpose(attn_c, (1, 0))


def _r2(a):
    return a.reshape(1, -1)


def kernel(x, mask, haltingscore, params):
    B, T, D = x.shape
    f32 = jnp.float32
    p1, p2 = params['dlwt'], params['dgwt']

    thr_r, wise, winid, calw_r = pl.pallas_call(
        _windows_kernel,
        out_shape=(
            jax.ShapeDtypeStruct((B, 128), f32),
            jax.ShapeDtypeStruct((B, T), f32),
            jax.ShapeDtypeStruct((B, T), f32),
            jax.ShapeDtypeStruct((B, 128), f32),
        ),
    )(haltingscore.astype(f32), mask.astype(f32))
    thr = thr_r[:, 0]
    wid3 = winid.reshape(B, 1, T)
    wise3 = wise.reshape(B, 1, T)

    xf = x.reshape(B * T, D)
    nflat = (B * T) // PTILE
    flat_spec = pl.BlockSpec((PTILE, D), lambda i: (i, 0))
    w_spec = pl.BlockSpec((D, D), lambda i: (0, 0))
    vec_spec = pl.BlockSpec((1, D), lambda i: (0, 0))
    xn, q1, k1, v1 = pl.pallas_call(
        _lnqkv_kernel,
        grid=(nflat,),
        in_specs=[flat_spec, vec_spec, vec_spec, w_spec, w_spec, w_spec],
        out_specs=(flat_spec,) * 4,
        out_shape=(jax.ShapeDtypeStruct((B * T, D), f32),) * 4,
    )(xf, _r2(params['ln1_g']), _r2(params['ln1_b']),
      p1['Wq'], p1['Wk'], p1['Wv'])

    nq = T // BQ
    qt_spec = pl.BlockSpec((1, BQ, D), lambda b, i: (b, i, 0))
    kv_spec = pl.BlockSpec((1, T, D), lambda b, i: (b, 0, 0))
    row_spec = pl.BlockSpec((1, 1, T), lambda b, i: (b, 0, 0))
    scr = pltpu.VMEM((BQ, N_HEAD * T), f32)
    ctx1, cs3 = pl.pallas_call(
        _attn1_kernel,
        grid=(B, nq),
        in_specs=[qt_spec, kv_spec, kv_spec, row_spec],
        out_specs=(qt_spec, row_spec),
        out_shape=(jax.ShapeDtypeStruct((B, T, D), f32),
                   jax.ShapeDtypeStruct((B, 1, T), f32)),
        scratch_shapes=[scr],
    )(q1.reshape(B, T, D), k1.reshape(B, T, D), v1.reshape(B, T, D), wid3)

    nt = T // PTILE
    tt_spec = pl.BlockSpec((1, PTILE, D), lambda b, i: (b, i, 0))
    vec2_spec = pl.BlockSpec((1, D), lambda b, i: (0, 0))
    wff1_spec = pl.BlockSpec((D, FFNDIM), lambda b, i: (0, 0))
    bff1_spec = pl.BlockSpec((1, FFNDIM), lambda b, i: (0, 0))
    wff2_spec = pl.BlockSpec((FFNDIM, D), lambda b, i: (0, 0))
    wo_spec = pl.BlockSpec((D, D), lambda b, i: (0, 0))
    lx, y = pl.pallas_call(
        _post1_kernel,
        grid=(B, nt),
        in_specs=[tt_spec, tt_spec, row_spec, row_spec,
                  wo_spec, vec2_spec, vec2_spec, wff1_spec, bff1_spec,
                  wff2_spec, vec2_spec, vec2_spec, vec2_spec],
        out_specs=(tt_spec, tt_spec),
        out_shape=(jax.ShapeDtypeStruct((B, T, D), f32),) * 2,
    )(ctx1, xn.reshape(B, T, D), wise3, cs3,
      p1['Wo'], _r2(p1['g1']), _r2(p1['b1']), p1['W1'], _r2(p1['bf1']),
      p1['W2'], _r2(p1['bf2']), _r2(p1['g2']), _r2(p1['b2']))

    full_spec = pl.BlockSpec((1, T, D), lambda b: (b, 0, 0))
    row1_spec = pl.BlockSpec((1, 1, T), lambda b: (b, 0, 0))
    nwt = T // PW
    pw_spec = pl.BlockSpec((1, PW, D), lambda b, i: (b, i, 0))
    yfull_spec = pl.BlockSpec((1, T, D), lambda b, i: (b, 0, 0))
    calw2_spec = pl.BlockSpec((B, 128), lambda b, i: (0, 0))
    pre, q2, k2, v2 = pl.pallas_call(
        _pool_kernel,
        grid=(B, nwt),
        in_specs=[yfull_spec, row_spec, calw2_spec, vec2_spec, vec2_spec,
                  wo_spec, wo_spec, wo_spec],
        out_specs=(pw_spec,) * 4,
        out_shape=(jax.ShapeDtypeStruct((B, T, D), f32),) * 4,
    )(y, wid3, calw_r, _r2(params['ln2_g']), _r2(params['ln2_b']),
      p2['Wq'], p2['Wk'], p2['Wv'])

    calw_spec = pl.BlockSpec((B, 128), lambda b, i: (0, 0))
    ctx2, hh3 = pl.pallas_call(
        _attn2_kernel,
        grid=(B, nq),
        in_specs=[qt_spec, kv_spec, kv_spec, calw_spec],
        out_specs=(qt_spec, row_spec),
        out_shape=(jax.ShapeDtypeStruct((B, T, D), f32),
                   jax.ShapeDtypeStruct((B, 1, T), f32)),
        scratch_shapes=[scr],
    )(q2, k2, v2, calw_r)

    gx = pl.pallas_call(
        _post2_kernel,
        grid=(B, nt),
        in_specs=[tt_spec, tt_spec, calw2_spec,
                  wo_spec, vec2_spec, vec2_spec, wff1_spec, bff1_spec,
                  wff2_spec, vec2_spec, vec2_spec, vec2_spec],
        out_specs=tt_spec,
        out_shape=jax.ShapeDtypeStruct((B, T, D), f32),
    )(ctx2, pre, calw_r,
      p2['Wo'], _r2(p2['g1']), _r2(p2['b1']), p2['W1'], _r2(p2['bf1']),
      p2['W2'], _r2(p2['bf2']), _r2(p2['g2']), _r2(p2['b2']))

    calwb_spec = pl.BlockSpec((B, 128), lambda b: (0, 0))
    data, attn3 = pl.pallas_call(
        _final_kernel,
        grid=(B,),
        in_specs=[full_spec, full_spec, row1_spec, row1_spec, row1_spec,
                  calwb_spec],
        out_specs=(full_spec, row1_spec),
        out_shape=(jax.ShapeDtypeStruct((B, T, D), f32),
                   jax.ShapeDtypeStruct((B, 1, T), f32)),
        scratch_shapes=[pltpu.VMEM((T, 2 * D), f32)],
    )(lx, gx, wid3, hh3, cs3, calw_r)

    return data, thr, attn3.reshape(B, T)
